# Initial kernel scaffold; baseline (speedup 1.0000x reference)
import jax
import jax.numpy as jnp
from jax.experimental import pallas as pl


def kernel(x, bias_values):
    raise NotImplementedError("write your pallas kernel here")


# speedup vs baseline: 41.6170x; 41.6170x over previous
import functools

import jax
import jax.numpy as jnp
from jax import lax
from jax.experimental import pallas as pl
from jax.experimental.pallas import tpu as pltpu
from jax.experimental.pallas import tpu_sc as plsc

_MAX_DIST = 1000
_NSHIFT = 16
_NUM_CORES = 2
_NUM_SUBCORES = 16


@functools.lru_cache(maxsize=None)
def _build_sc_kernel(S, H, VLEN):
    NW = _NUM_CORES * _NUM_SUBCORES
    ROWS = H * S
    RPW = ROWS // NW
    assert ROWS % NW == 0 and S % RPW == 0, (S, H)
    W = 2 * S
    assert VLEN == W + _NSHIFT
    CH = 16

    mesh = plsc.VectorSubcoreMesh(
        core_axis_name="c", subcore_axis_name="s",
        num_cores=_NUM_CORES, num_subcores=_NUM_SUBCORES)

    @functools.partial(
        pl.kernel,
        out_type=jax.ShapeDtypeStruct((ROWS * S,), jnp.float32),
        mesh=mesh,
        scratch_types=(
            [pltpu.VMEM((VLEN,), jnp.float32)]
            + [pltpu.VMEM((W,), jnp.float32)] * _NSHIFT
            + [pltpu.SemaphoreType.DMA]
        ),
    )
    def sc_kernel(v_hbm, out_hbm, vsrc, *rest):
        vs = rest[:_NSHIFT]
        sem = rest[_NSHIFT]
        wid = lax.axis_index("s") * _NUM_CORES + lax.axis_index("c")
        r0 = wid * RPW
        h = r0 // S
        i0 = r0 - h * S

        pltpu.sync_copy(v_hbm.at[h], vsrc)

        for m in range(_NSHIFT):
            def shift_body(kc, _, m=m):
                vs[m][pl.ds(kc * CH, CH)] = vsrc[pl.ds(m + kc * CH, CH)]
                return _
            lax.fori_loop(0, W // CH, shift_body, None)

        for m in range(_NSHIFT):
            o = (S - 1 - m) % _NSHIFT

            def fire(t, _, m=m, o=o):
                i = i0 + o + t * _NSHIFT
                a = pl.multiple_of((S - 1) - i - m, _NSHIFT)
                pltpu.make_async_copy(
                    vs[m].at[pl.ds(a, S)],
                    out_hbm.at[pl.ds((r0 + o + t * _NSHIFT) * S, S)],
                    sem).start()
                return _
            lax.fori_loop(0, RPW // _NSHIFT, fire, None)

        def drain(t, _):
            pltpu.make_async_copy(
                vs[0].at[pl.ds(0, S)], out_hbm.at[pl.ds(r0 * S, S)],
                sem).wait()
            return _
        lax.fori_loop(0, RPW, drain, None)

    return sc_kernel


def kernel(x, bias_values):
    S = x.shape[1]
    NB, H = bias_values.shape
    assert NB == 2 * _MAX_DIST
    VLEN = 2 * S + _NSHIFT
    n_left = (S - 1) - _MAX_DIST
    n_right = VLEN - n_left - NB
    assert n_left >= 0 and n_right >= 1

    v = jnp.concatenate([
        jnp.broadcast_to(bias_values[0], (n_left, H)),
        bias_values,
        jnp.broadcast_to(bias_values[-1], (n_right, H)),
    ], axis=0).T

    out = _build_sc_kernel(S, H, VLEN)(v)
    return out.reshape(1, H, S, S)
